# Initial kernel scaffold; baseline (speedup 1.0000x reference)
import jax
import jax.numpy as jnp
from jax.experimental import pallas as pl


def kernel(x_slot, x_domain, edge_index_slot, edge_index_domain, Wq_s, bq_s, Wk_s, bk_s, Wv_s, bv_s, Wo_s, bo_s, Wq_d, bq_d, Wk_d, bk_d, Wv_d, bv_d, Wo_d, bo_d):
    raise NotImplementedError("write your pallas kernel here")


# speedup vs baseline: 36.6408x; 36.6408x over previous
import functools

import jax
import jax.numpy as jnp
from jax import lax
from jax.experimental import pallas as pl
from jax.experimental.pallas import tpu as pltpu
from jax.experimental.pallas import tpu_sc as plsc

N = 10000
E = 320000
D = 128
H = 8
HD = D // H

NC = 2
NS = 16
NW = NC * NS
EPW = E // NW
C = 80
CHUNKS = EPW // C
SS = 80
NSTR = N // SS
TMAX = -(-NSTR // NS)
DR = 1280

_mesh = plsc.VectorSubcoreMesh(core_axis_name="c", subcore_axis_name="s",
                               num_cores=NC, num_subcores=NS)

_sc_params = pltpu.CompilerParams(needs_layout_passes=False)

_f32 = jnp.float32


@functools.partial(
    pl.kernel,
    out_type=[jax.ShapeDtypeStruct((E * 16,), _f32),
              jax.ShapeDtypeStruct((NW * 16,), _f32)],
    mesh=_mesh,
    scratch_types=[
        pltpu.VMEM((C,), jnp.int32),
        pltpu.VMEM((C,), jnp.int32),
        pltpu.VMEM((C, D), _f32),
        pltpu.VMEM((C, D), _f32),
        pltpu.VMEM((C * 16,), _f32),
        pltpu.VMEM((16,), _f32),
        pltpu.SemaphoreType.DMA,
    ],
    compiler_params=_sc_params,
)
def _sc_scores(q_hbm, k_hbm, dst_hbm, src_hbm, scores_hbm, maxp_hbm,
               didx, sidx, qrows, krows, sscr, mscr, sem):
  c = lax.axis_index("c")
  s = lax.axis_index("s")
  wid = s * NC + c
  base = wid * EPW

  zv = jnp.zeros((16,), _f32)
  mscr[pl.ds(0, 16)] = zv
  lane = lax.iota(jnp.int32, 16)
  onehot = [(lane == h).astype(_f32) for h in range(H)]

  @pl.loop(0, CHUNKS)
  def _chunk(i):
    off = base + i * C
    pltpu.sync_copy(dst_hbm.at[pl.ds(off, C)], didx)
    pltpu.sync_copy(src_hbm.at[pl.ds(off, C)], sidx)
    cq = pltpu.async_copy(q_hbm.at[didx], qrows, sem)
    ck = pltpu.async_copy(k_hbm.at[sidx], krows, sem)
    cq.wait()
    ck.wait()

    @pl.loop(0, C)
    def _edge(e):
      row = zv
      for h in range(H):
        qv = qrows[e, pl.ds(h * HD, 16)]
        kv = krows[e, pl.ds(h * HD, 16)]
        row = row + jnp.sum(qv * kv) * onehot[h]
      sscr[pl.ds(e * 16, 16)] = row
      mscr[pl.ds(0, 16)] = jnp.maximum(mscr[pl.ds(0, 16)], row)

    pltpu.sync_copy(sscr, scores_hbm.at[pl.ds(off * 16, C * 16)])

  pltpu.sync_copy(mscr, maxp_hbm.at[pl.ds(wid * 16, 16)])


@functools.partial(
    pl.kernel,
    out_type=[jax.ShapeDtypeStruct((NC * N, D), _f32),
              jax.ShapeDtypeStruct((NC * DR, D), _f32),
              jax.ShapeDtypeStruct((NC * N, D), _f32),
              jax.ShapeDtypeStruct((NC * DR, D), _f32)],
    mesh=_mesh,
    scratch_types=[
        pltpu.VMEM((C,), jnp.int32),
        pltpu.VMEM((C,), jnp.int32),
        pltpu.VMEM((C,), jnp.int32),
        pltpu.VMEM((C, D), _f32),
        pltpu.VMEM((C * 16,), _f32),
        pltpu.VMEM((C, D), _f32),
        pltpu.VMEM((NW * 16,), _f32),
        pltpu.VMEM_SHARED((N, D), _f32),
        pltpu.VMEM_SHARED((DR, D), _f32),
        pltpu.SemaphoreType.DMA,
    ],
    compiler_params=_sc_params,
)
def _sc_accum(v0_hbm, scores0_hbm, maxp0_hbm, dst0_hbm, src0_hbm,
              v1_hbm, scores1_hbm, maxp1_hbm, dst1_hbm, src1_hbm,
              outp0_hbm, denp0_hbm, outp1_hbm, denp1_hbm,
              didx, didx2, sidx, vrows, sscr, escr, gall, out_acc, den_acc,
              sem):
  c = lax.axis_index("c")
  s = lax.axis_index("s")
  wid = s * NC + c
  base = wid * EPW
  lane = lax.iota(jnp.int32, 16)
  mask = (lane < 8).astype(_f32)
  zv = jnp.zeros((16,), _f32)

  @pl.loop(0, C)
  def _ze(r):
    for k in range(D // 16):
      escr[r, pl.ds(k * 16, 16)] = zv

  def run_domain(v_hbm, scores_hbm, maxp_hbm, dst_hbm, src_hbm,
                 outp_hbm, denp_hbm):
    pltpu.sync_copy(maxp_hbm, gall)
    g = gall[pl.ds(0, 16)]
    for w in range(1, NW):
      g = jnp.maximum(g, gall[pl.ds(w * 16, 16)])

    @pl.loop(0, C)
    def _zz(r):
      for k in range(D // 16):
        vrows[r, pl.ds(k * 16, 16)] = zv

    for t in range(TMAX):
      stripe = t * NS + s
      @pl.when(stripe < NSTR)
      def _zs():
        pltpu.sync_copy(vrows, out_acc.at[pl.ds(stripe * SS, SS)])
    pltpu.sync_copy(vrows, den_acc.at[pl.ds(s * SS, SS)])

    plsc.subcore_barrier()

    @pl.loop(0, CHUNKS)
    def _chunk(i):
      off = base + i * C
      pltpu.sync_copy(dst_hbm.at[pl.ds(off, C)], didx)
      pltpu.sync_copy(src_hbm.at[pl.ds(off, C)], sidx)
      cv = pltpu.async_copy(v_hbm.at[sidx], vrows, sem)
      pltpu.sync_copy(scores_hbm.at[pl.ds(off * 16, C * 16)], sscr)

      @pl.loop(0, C // 16)
      def _ex(grp):
        dstv = didx[pl.ds(grp * 16, 16)]
        didx2[pl.ds(grp * 16, 16)] = lax.shift_right_logical(dstv, 3)
        for j in range(16):
          e = grp * 16 + j
          slot = (dstv[j] & 7) * 16
          sv = sscr[pl.ds(e * 16, 16)]
          escr[e, pl.ds(slot, 16)] = jnp.exp(sv - g) * mask

      pltpu.sync_copy(escr, den_acc.at[didx2], add=True)

      cv.wait()

      @pl.loop(0, C // 16)
      def _scale(grp):
        dstv = didx[pl.ds(grp * 16, 16)]
        for j in range(16):
          e = grp * 16 + j
          slot = (dstv[j] & 7) * 16
          exv = escr[e, pl.ds(slot, 16)]
          for h in range(H):
            vv = vrows[e, pl.ds(h * HD, 16)]
            vrows[e, pl.ds(h * HD, 16)] = vv * exv[h]
          escr[e, pl.ds(slot, 16)] = zv

      pltpu.sync_copy(vrows, out_acc.at[didx], add=True)

    plsc.subcore_barrier()

    for t in range(TMAX):
      stripe = t * NS + s
      @pl.when(stripe < NSTR)
      def _flush():
        r = stripe * SS
        pltpu.sync_copy(out_acc.at[pl.ds(r, SS)],
                        outp_hbm.at[pl.ds(c * N + r, SS)])
    pltpu.sync_copy(den_acc.at[pl.ds(s * SS, SS)],
                    denp_hbm.at[pl.ds(c * DR + s * SS, SS)])

    plsc.subcore_barrier()

  run_domain(v0_hbm, scores0_hbm, maxp0_hbm, dst0_hbm, src0_hbm,
             outp0_hbm, denp0_hbm)
  run_domain(v1_hbm, scores1_hbm, maxp1_hbm, dst1_hbm, src1_hbm,
             outp1_hbm, denp1_hbm)


def _qkv_body(x_ref, wq, wk, wv, bq, bk, bv, q_out, k_out, v_out):
  xb = x_ref[...]
  q_out[...] = (jnp.dot(xb, wq[...], preferred_element_type=_f32)
                + bq[...]) * (1.0 / (HD ** 0.5))
  k_out[...] = jnp.dot(xb, wk[...], preferred_element_type=_f32) + bk[...]
  v_out[...] = jnp.dot(xb, wv[...], preferred_element_type=_f32) + bv[...]


_BR = 1000
_NBLK = N // _BR

_qkv_call = pl.pallas_call(
    _qkv_body,
    grid=(_NBLK,),
    in_specs=[
        pl.BlockSpec((_BR, D), lambda i: (i, 0)),
        pl.BlockSpec((D, D), lambda i: (0, 0)),
        pl.BlockSpec((D, D), lambda i: (0, 0)),
        pl.BlockSpec((D, D), lambda i: (0, 0)),
        pl.BlockSpec((1, D), lambda i: (0, 0)),
        pl.BlockSpec((1, D), lambda i: (0, 0)),
        pl.BlockSpec((1, D), lambda i: (0, 0)),
    ],
    out_specs=[
        pl.BlockSpec((_BR, D), lambda i: (i, 0)),
        pl.BlockSpec((_BR, D), lambda i: (i, 0)),
        pl.BlockSpec((_BR, D), lambda i: (i, 0)),
    ],
    out_shape=[jax.ShapeDtypeStruct((N, D), _f32)] * 3,
)


def _final_body(p0, p1, d0, d1, sel, wo, bo, x_ref, o_ref):
  msg = p0[...] + p1[...]
  den = d0[...][:, :H] + d1[...][:, :H] + 1e-16
  dexp = jnp.dot(1.0 / den, sel[...], preferred_element_type=_f32)
  y = jnp.dot(msg * dexp, wo[...], preferred_element_type=_f32)
  o_ref[...] = y + bo[...] + x_ref[...]


_final_call = pl.pallas_call(
    _final_body,
    grid=(_NBLK,),
    in_specs=[
        pl.BlockSpec((_BR, D), lambda i: (i, 0)),
        pl.BlockSpec((_BR, D), lambda i: (i + _NBLK, 0)),
        pl.BlockSpec((_BR, 16), lambda i: (i, 0)),
        pl.BlockSpec((_BR, 16), lambda i: (i, 0)),
        pl.BlockSpec((H, D), lambda i: (0, 0)),
        pl.BlockSpec((D, D), lambda i: (0, 0)),
        pl.BlockSpec((1, D), lambda i: (0, 0)),
        pl.BlockSpec((_BR, D), lambda i: (i, 0)),
    ],
    out_specs=pl.BlockSpec((_BR, D), lambda i: (i, 0)),
    out_shape=jax.ShapeDtypeStruct((N, D), _f32),
)


def _unpack_den(denp):
  dv = denp.reshape(NC, DR * D // 16, 16)
  return dv[0, :N], dv[1, :N]


def kernel(x_slot, x_domain, edge_index_slot, edge_index_domain,
           Wq_s, bq_s, Wk_s, bk_s, Wv_s, bv_s, Wo_s, bo_s,
           Wq_d, bq_d, Wk_d, bk_d, Wv_d, bv_d, Wo_d, bo_d):
  lane = jnp.arange(D, dtype=jnp.int32)
  head = jnp.arange(H, dtype=jnp.int32)
  sel = (lane[None, :] // HD == head[:, None]).astype(_f32)

  src_s, dst_s = edge_index_slot[0], edge_index_slot[1]
  src_d, dst_d = edge_index_domain[0], edge_index_domain[1]

  q_s, k_s, v_s = _qkv_call(x_slot, Wq_s, Wk_s, Wv_s, bq_s.reshape(1, D),
                            bk_s.reshape(1, D), bv_s.reshape(1, D))
  q_d, k_d, v_d = _qkv_call(x_domain, Wq_d, Wk_d, Wv_d, bq_d.reshape(1, D),
                            bk_d.reshape(1, D), bv_d.reshape(1, D))

  scores_s, maxp_s = _sc_scores(q_s, k_s, dst_s, src_s)
  scores_d, maxp_d = _sc_scores(q_d, k_d, dst_d, src_d)

  outp_s, denp_s, outp_d, denp_d = _sc_accum(
      v_s, scores_s, maxp_s, dst_s, src_s,
      v_d, scores_d, maxp_d, dst_d, src_d)

  d0_s, d1_s = _unpack_den(denp_s)
  d0_d, d1_d = _unpack_den(denp_d)

  slot_out = _final_call(outp_s, outp_s, d0_s, d1_s, sel, Wo_s,
                         bo_s.reshape(1, D), x_slot)
  dom_out = _final_call(outp_d, outp_d, d0_d, d1_d, sel, Wo_d,
                        bo_d.reshape(1, D), x_domain)
  return slot_out, dom_out
